# Initial kernel scaffold; baseline (speedup 1.0000x reference)
import jax
import jax.numpy as jnp
from jax.experimental import pallas as pl


def kernel(label_2d, vertex_pred, extents, poses, meta_data):
    raise NotImplementedError("write your pallas kernel here")


# speedup vs baseline: 1.1543x; 1.1543x over previous
import functools

import jax
import jax.numpy as jnp
from jax import lax
from jax.experimental import pallas as pl
from jax.experimental.pallas import tpu as pltpu
from jax.experimental.pallas import tpu_sc as plsc

NUM_CLASSES = 22
SKIP = 20
N_STEPS = 64
STEP_SIZE = 8.0
H, W = 480, 640
HW = H * W
P = HW // SKIP
PROWS = P // 128
GRID = PROWS // 8
GRPS = ((0, 6), (6, 11), (11, 17), (17, 22))
GSZ = tuple((b - a) * HW for a, b in GRPS)
VOTES_PER_GROUP = P * N_STEPS
VPT = VOTES_PER_GROUP // 16
CH = 5120
AB = HW // 16
WB = 4800


def _stage1_body(labs_ref, vpt_ref, labf_ref, votes_ref, stats_ref):
    r = pl.program_id(0)

    @pl.when(r == 0)
    def _():
        stats_ref[...] = jnp.zeros((72, 128), jnp.float32)

    lab = labs_ref[...]
    labf = labf_ref[...]
    sub = lax.broadcasted_iota(jnp.int32, (8, 128), 0)
    lane = lax.broadcasted_iota(jnp.int32, (8, 128), 1)
    i = (r * 8 + sub) * 128 + lane
    ys = (i // 32).astype(jnp.float32)
    xs = (20 * (i % 32)).astype(jnp.float32)

    dx = jnp.zeros((8, 128), jnp.float32)
    dy = jnp.zeros((8, 128), jnp.float32)
    dz = jnp.zeros((8, 128), jnp.float32)
    for c in range(NUM_CLASSES):
        m = lab == c
        dx = jnp.where(m, vpt_ref[3 * c + 0], dx)
        dy = jnp.where(m, vpt_ref[3 * c + 1], dy)
        dz = jnp.where(m, vpt_ref[3 * c + 2], dz)
        mf = jnp.sum(m.astype(jnp.float32), axis=0)
        stats_ref[c] = stats_ref[c] + mf
        full = jnp.sum((labf == c).astype(jnp.float32), axis=0)
        stats_ref[22 + c] = stats_ref[22 + c] + full
        zsum = jnp.sum(jnp.where(m, dz, 0.0), axis=0)
        stats_ref[44 + c] = stats_ref[44 + c] + zsum

    nrm = jnp.sqrt(dx * dx + dy * dy) + 1e-6
    ux = dx / nrm
    uy = dy / nrm
    lab_pos = lab > 0
    flat_base = lab * HW
    for s in range(N_STEPS):
        step = (s + 1) * STEP_SIZE
        cx = xs + ux * step
        cy = ys + uy * step
        valid = (cx >= 0) & (cx <= W - 1) & (cy >= 0) & (cy <= H - 1) & lab_pos
        cxi = jnp.clip(jnp.round(cx), 0, W - 1).astype(jnp.int32)
        cyi = jnp.clip(jnp.round(cy), 0, H - 1).astype(jnp.int32)
        flat = flat_base + cyi * W + cxi
        for g in range(4):
            c0, c1 = GRPS[g]
            ing = valid & (lab >= c0) & (lab < c1)
            votes_ref[g, s] = jnp.where(ing, flat - c0 * HW, GSZ[g])


def _stage2_body(votes_hbm, maxv_hbm, amax_hbm, acc, idxb, ones, wbuf,
                 mstage, istage):
    k = lax.axis_index("c")
    s = lax.axis_index("s")
    lanes = lax.iota(jnp.int32, 16)

    def initloop(i, _):
        ones[pl.ds(i * 16, 16)] = jnp.ones((16,), jnp.float32)
        return 0

    lax.fori_loop(0, CH // 16, initloop, 0)

    for p in range(2):
        ncls = 6 - p
        sz = ncls * HW
        c0 = 11 * k + 6 * p
        g = 2 * k + p

        def zloop(i, _):
            wbuf[pl.ds(i * 16, 16)] = jnp.zeros((16,), jnp.float32)
            return 0

        lax.fori_loop(0, WB // 16, zloop, 0)
        per_tile = sz // 16
        for j in range(per_tile // WB):
            pltpu.sync_copy(wbuf, acc.at[pl.ds(s * per_tile + j * WB, WB)])

        @pl.when(s == 0)
        def _():
            pltpu.sync_copy(wbuf.at[pl.ds(0, 64)], acc.at[pl.ds(sz, 64)])

        plsc.subcore_barrier()

        base = g * VOTES_PER_GROUP + s * VPT
        for j in range(VPT // CH):
            pltpu.sync_copy(votes_hbm.at[pl.ds(base + j * CH, CH)], idxb)
            pltpu.sync_copy(ones, acc.at[idxb], add=True)
        plsc.subcore_barrier()

        for cl in range(ncls):
            cm = jnp.full((16,), -1.0, jnp.float32)
            ci = jnp.zeros((16,), jnp.int32)
            for ch in range(AB // WB):
                pltpu.sync_copy(
                    acc.at[pl.ds(cl * HW + s * AB + ch * WB, WB)], wbuf)
                ibase = s * AB + ch * WB

                def amloop(i, carry, ibase=ibase):
                    cm, ci = carry
                    v = wbuf[pl.ds(i * 16, 16)]
                    mgt = v > cm
                    cm = jnp.where(mgt, v, cm)
                    ci = jnp.where(mgt, lanes + (ibase + i * 16), ci)
                    return (cm, ci)

                cm, ci = lax.fori_loop(0, WB // 16, amloop, (cm, ci))
            mstage[...] = cm
            istage[...] = ci
            pltpu.sync_copy(mstage, maxv_hbm.at[c0 + cl, s])
            pltpu.sync_copy(istage, amax_hbm.at[c0 + cl, s])
        plsc.subcore_barrier()


def _stage3_body(stats_ref, maxv_ref, amax_ref, ext_ref, meta_ref, out_ref):
    stats = stats_ref[...]
    counts_sel = jnp.sum(stats[0:22], axis=1, keepdims=True)
    counts_full = jnp.sum(stats[22:44], axis=1, keepdims=True)
    sum_dz = jnp.sum(stats[44:66], axis=1, keepdims=True)
    mv = maxv_ref[...]
    am = amax_ref[...]
    gmax = jnp.max(mv, axis=1, keepdims=True)
    idxs = jnp.where(mv == gmax, am, jnp.int32(2 ** 30))
    gidx = jnp.min(idxs, axis=1, keepdims=True)
    gf = gidx.astype(jnp.float32)
    cyc = jnp.floor(gf / W)
    cxc = gf - cyc * W
    mean_dz = sum_dz / jnp.maximum(counts_sel, 1.0)
    z = jnp.exp(jnp.clip(mean_dz, -3.0, 3.0))
    ext = ext_ref[...]
    extn = jnp.sqrt(jnp.sum(ext * ext, axis=1, keepdims=True))
    fx = meta_ref[0, 0]
    fy = meta_ref[0, 4]
    px = meta_ref[0, 2]
    py = meta_ref[0, 5]
    bw = extn * fx / z
    bh = extn * fy / z
    clsi = lax.broadcasted_iota(jnp.int32, (22, 1), 0)
    clsf = clsi.astype(jnp.float32)
    is_valid = ((counts_full > 500.0) & (gmax > 0.02 * counts_sel)
                & (clsi > 0)).astype(jnp.float32)
    x1 = cxc - bw * 0.5
    y1 = cyc - bh * 0.5
    x2 = cxc + bw * 0.5
    y2 = cyc + bh * 0.5
    tx = (cxc - px) * z / fx
    ty = (cyc - py) * z / fy
    zero = jnp.zeros((22, 1), jnp.float32)
    one = jnp.ones((22, 1), jnp.float32)
    cols = [zero, clsf, x1, y1, x2, y2, gmax,
            zero, clsf, one, zero, zero, zero, tx, ty, z, zero, zero, zero,
            zero]
    out_ref[...] = jnp.concatenate(cols, axis=1) * is_valid


@functools.lru_cache(maxsize=1)
def _make_stage2():
    mesh = plsc.VectorSubcoreMesh(core_axis_name="c", subcore_axis_name="s")
    return pl.kernel(
        _stage2_body,
        out_type=(
            jax.ShapeDtypeStruct((NUM_CLASSES, 16, 16), jnp.float32),
            jax.ShapeDtypeStruct((NUM_CLASSES, 16, 16), jnp.int32),
        ),
        mesh=mesh,
        scratch_types=[
            pltpu.VMEM_SHARED((6 * HW + 64,), jnp.float32),
            pltpu.VMEM((CH,), jnp.int32),
            pltpu.VMEM((CH,), jnp.float32),
            pltpu.VMEM((WB,), jnp.float32),
            pltpu.VMEM((16,), jnp.float32),
            pltpu.VMEM((16,), jnp.int32),
        ],
    )


@jax.jit
def kernel(label_2d, vertex_pred, extents, poses, meta_data):
    del poses
    C = NUM_CLASSES
    lab_flat = label_2d[0].reshape(P, SKIP)[:, 0]
    labs = lab_flat.reshape(PROWS, 128)
    vp_s = vertex_pred[0].reshape(P, SKIP, 3 * C)[:, 0, :]
    vpt = (vp_s.reshape(PROWS, 128, C, 3)
           .transpose(2, 3, 0, 1)
           .reshape(3 * C, PROWS, 128))
    lab_full = label_2d[0].reshape(2400, 128)

    votes, stats = pl.pallas_call(
        _stage1_body,
        grid=(GRID,),
        in_specs=[
            pl.BlockSpec((8, 128), lambda r: (r, 0)),
            pl.BlockSpec((3 * C, 8, 128), lambda r: (0, r, 0)),
            pl.BlockSpec((160, 128), lambda r: (r, 0)),
        ],
        out_specs=[
            pl.BlockSpec((4, N_STEPS, 8, 128), lambda r: (0, 0, r, 0)),
            pl.BlockSpec((72, 128), lambda r: (0, 0)),
        ],
        out_shape=[
            jax.ShapeDtypeStruct((4, N_STEPS, PROWS, 128), jnp.int32),
            jax.ShapeDtypeStruct((72, 128), jnp.float32),
        ],
    )(labs, vpt, lab_full)

    votes_flat = votes.reshape(4 * VOTES_PER_GROUP)
    maxv, amax = _make_stage2()(votes_flat)

    ext_p = jnp.pad(extents, ((0, 0), (0, 128 - 3)))
    meta_p = jnp.pad(meta_data, ((0, 7), (0, 128 - 48)))
    out = pl.pallas_call(
        _stage3_body,
        out_shape=jax.ShapeDtypeStruct((C, 20), jnp.float32),
    )(stats, maxv.reshape(C, 256), amax.reshape(C, 256), ext_p, meta_p)
    return out
